# baseline (speedup 1.0000x reference)
import jax
import jax.numpy as jnp
from jax.experimental import pallas as pl

N = 50000
H = 4
D = 32
HD = H * D


def _gat_layer(h, src, dst, W, al, ar, b):
    feat = (h @ W).reshape(-1, H, D)
    el = jnp.sum(feat * al[None, :, :], axis=-1)
    er = jnp.sum(feat * ar[None, :, :], axis=-1)
    e = jax.nn.leaky_relu(el[src] + er[dst], negative_slope=0.2)
    emax = jax.ops.segment_max(e, dst, num_segments=N)
    ee = jnp.exp(e - emax[dst])
    esum = jax.ops.segment_sum(ee, dst, num_segments=N)
    alpha = ee / esum[dst]
    out = jax.ops.segment_sum(alpha[:, :, None] * feat[src], dst, num_segments=N)
    return out + b.reshape(1, H, D)


def _mlp_body(h_ref, wd1_ref, bd1_ref, wd2_ref, bd2_ref, out_ref, acc_ref):
    i = pl.program_id(0)
    h = jnp.maximum(h_ref[...], 0.0)
    y = jnp.maximum(h @ wd1_ref[...] + bd1_ref[...], 0.0)
    o = y @ wd2_ref[...] + bd2_ref[...] + 0.5
    out_ref[...] = o

    @pl.when(i == 0)
    def _():
        acc_ref[...] = jnp.zeros_like(acc_ref)

    acc_ref[...] += jnp.sum(o, axis=0, keepdims=True) / N


def kernel(seq, node_s, edge_index, W_s, W0, al0, ar0, b0, W1, al1, ar1, b1,
           W2, al2, ar2, b2, Wd1, bd1, Wd2, bd2):
    src = edge_index[0]
    dst = edge_index[1]
    emb = jnp.take(W_s, seq, axis=0)
    h = jnp.concatenate([emb, node_s], axis=1)
    h = _gat_layer(h, src, dst, W0, al0, ar0, b0).reshape(-1, HD)
    h = _gat_layer(h, src, dst, W1, al1, ar1, b1).reshape(-1, HD)
    h = _gat_layer(h, src, dst, W2, al2, ar2, b2).reshape(-1, HD)
    BN = 1000
    out, graph = pl.pallas_call(
        _mlp_body,
        grid=(N // BN,),
        in_specs=[
            pl.BlockSpec((BN, HD), lambda i: (i, 0)),
            pl.BlockSpec((HD, 512), lambda i: (0, 0)),
            pl.BlockSpec((512,), lambda i: (0,)),
            pl.BlockSpec((512, 1), lambda i: (0, 0)),
            pl.BlockSpec((1,), lambda i: (0,)),
        ],
        out_specs=[
            pl.BlockSpec((BN, 1), lambda i: (i, 0)),
            pl.BlockSpec((1, 1), lambda i: (0, 0)),
        ],
        out_shape=[
            jax.ShapeDtypeStruct((N, 1), jnp.float32),
            jax.ShapeDtypeStruct((1, 1), jnp.float32),
        ],
    )(h, Wd1, bd1, Wd2, bd2)
    return out, graph


# speedup vs baseline: 28.8216x; 28.8216x over previous
import functools

import jax
import jax.numpy as jnp
from jax import lax
from jax.experimental import pallas as pl
from jax.experimental.pallas import tpu as pltpu
from jax.experimental.pallas import tpu_sc as plsc

N = 50000
E = 800000
H = 4
D = 32
HD = H * D

BN = 512
N_pad = 50176
E_pad = 819200
NBLK = N_pad // BN
NTILE = 32
EPT = E_pad // NTILE
CH = 512
NCHUNK = EPT // CH
ROWS_T = N_pad // 16
ZROWS = 392


def _fori(n, body):
    lax.fori_loop(0, n, lambda i, c: (body(i), 0)[1], 0)


_LOG2E = 1.4426950408889634
_LN2 = 0.6931471805599453
_RND = 12582912.0


def _exp_precise(x, tab_a, tab_b):
    x = jnp.minimum(jnp.maximum(x, -43.0), 43.0)
    t = x * _LOG2E
    nf = (t + _RND) - _RND
    z = (t - nf) * _LN2
    p = 1.0 / 5040.0
    for c in (1.0 / 720.0, 1.0 / 120.0, 1.0 / 24.0, 1.0 / 6.0, 0.5, 1.0, 1.0):
        p = p * z + c
    ni = nf.astype(jnp.int32)
    n1 = ni >> 3
    n2 = ni - (n1 << 3)
    ga = tab_a.at[n1 + 8].get(mode="promise_in_bounds")
    gb = tab_b.at[n2].get(mode="promise_in_bounds")
    return p * ga * gb


def _tc_first_body(seq_ref, ns_ref, ws_ref, w0_ref, al_ref, ar_ref,
                   feat_ref, el_ref, er_ref):
    t0 = jnp.dot(ws_ref[...], w0_ref[0:20, :], preferred_element_type=jnp.float32, precision=lax.Precision.HIGHEST)
    sv = seq_ref[...]
    iota = lax.broadcasted_iota(jnp.int32, (BN, 20), 1)
    oh = (sv == iota).astype(jnp.float32)
    feat = jnp.dot(oh, t0, preferred_element_type=jnp.float32, precision=lax.Precision.HIGHEST)
    feat = feat + jnp.dot(ns_ref[...], w0_ref[20:26, :],
                          preferred_element_type=jnp.float32, precision=lax.Precision.HIGHEST)
    el_cols = []
    er_cols = []
    for h in range(H):
        fh = feat[:, 32 * h:32 * h + 32]
        feat_ref[h] = fh
        el_cols.append(jnp.sum(fh * al_ref[h, :][None, :], axis=1, keepdims=True))
        er_cols.append(jnp.sum(fh * ar_ref[h, :][None, :], axis=1, keepdims=True))
    z12 = jnp.zeros((BN, 12), jnp.float32)
    el_ref[...] = jnp.concatenate(el_cols + [z12], axis=1)
    er_ref[...] = jnp.concatenate(er_cols + [z12], axis=1)


def _tc_mid_body(u_ref, es_ref, b_ref, w_ref, al_ref, ar_ref,
                 feat_ref, el_ref, er_ref):
    es = es_ref[0, :, 0:4] + es_ref[1, :, 0:4]
    esc = jnp.where(es == 0.0, 1.0, es)
    feat = jnp.zeros((BN, HD), jnp.float32)
    for h in range(H):
        xh = u_ref[h] / esc[:, h:h + 1] + b_ref[h, :][None, :]
        feat = feat + jnp.dot(xh, w_ref[32 * h:32 * h + 32, :],
                              preferred_element_type=jnp.float32, precision=lax.Precision.HIGHEST)
    el_cols = []
    er_cols = []
    for h in range(H):
        fh = feat[:, 32 * h:32 * h + 32]
        feat_ref[h] = fh
        el_cols.append(jnp.sum(fh * al_ref[h, :][None, :], axis=1, keepdims=True))
        er_cols.append(jnp.sum(fh * ar_ref[h, :][None, :], axis=1, keepdims=True))
    z12 = jnp.zeros((BN, 12), jnp.float32)
    el_ref[...] = jnp.concatenate(el_cols + [z12], axis=1)
    er_ref[...] = jnp.concatenate(er_cols + [z12], axis=1)


def _tc_final_body(u_ref, es_ref, b_ref, wd1_ref, bd1_ref, wd2_ref, bd2_ref,
                   out_ref, acc_ref):
    i = pl.program_id(0)
    es = es_ref[0, :, 0:4] + es_ref[1, :, 0:4]
    esc = jnp.where(es == 0.0, 1.0, es)
    y = jnp.zeros((BN, 512), jnp.float32)
    for h in range(H):
        xh = u_ref[h] / esc[:, h:h + 1] + b_ref[h, :][None, :]
        xh = jnp.maximum(xh, 0.0)
        y = y + jnp.dot(xh, wd1_ref[32 * h:32 * h + 32, :],
                        preferred_element_type=jnp.float32, precision=lax.Precision.HIGHEST)
    y = jnp.maximum(y + bd1_ref[...], 0.0)
    o = jnp.dot(y, wd2_ref[...], preferred_element_type=jnp.float32, precision=lax.Precision.HIGHEST)
    o = o + bd2_ref[...] + 0.5
    out_ref[...] = o

    @pl.when(i == 0)
    def _():
        acc_ref[...] = jnp.zeros_like(acc_ref)

    rows = lax.broadcasted_iota(jnp.int32, (BN, 1), 0) + i * BN
    om = jnp.where(rows < N, o, 0.0)
    acc_ref[...] += jnp.sum(om, axis=(0, 1), keepdims=False).reshape(1, 1) / N


def _tc_first(seq2, ns_p, W_s, W0, al, ar):
    return pl.pallas_call(
        _tc_first_body,
        grid=(NBLK,),
        in_specs=[
            pl.BlockSpec((BN, 1), lambda i: (i, 0)),
            pl.BlockSpec((BN, 6), lambda i: (i, 0)),
            pl.BlockSpec((20, 20), lambda i: (0, 0)),
            pl.BlockSpec((26, HD), lambda i: (0, 0)),
            pl.BlockSpec((H, D), lambda i: (0, 0)),
            pl.BlockSpec((H, D), lambda i: (0, 0)),
        ],
        out_specs=[
            pl.BlockSpec((H, BN, D), lambda i: (0, i, 0)),
            pl.BlockSpec((BN, 16), lambda i: (i, 0)),
            pl.BlockSpec((BN, 16), lambda i: (i, 0)),
        ],
        out_shape=[
            jax.ShapeDtypeStruct((H, N_pad, D), jnp.float32),
            jax.ShapeDtypeStruct((N_pad, 16), jnp.float32),
            jax.ShapeDtypeStruct((N_pad, 16), jnp.float32),
        ],
    )(seq2, ns_p, W_s, W0, al, ar)


def _tc_mid(U, esum, b, W, al, ar):
    return pl.pallas_call(
        _tc_mid_body,
        grid=(NBLK,),
        in_specs=[
            pl.BlockSpec((H, BN, D), lambda i: (0, i, 0)),
            pl.BlockSpec((2, BN, 16), lambda i: (0, i, 0)),
            pl.BlockSpec((H, D), lambda i: (0, 0)),
            pl.BlockSpec((HD, HD), lambda i: (0, 0)),
            pl.BlockSpec((H, D), lambda i: (0, 0)),
            pl.BlockSpec((H, D), lambda i: (0, 0)),
        ],
        out_specs=[
            pl.BlockSpec((H, BN, D), lambda i: (0, i, 0)),
            pl.BlockSpec((BN, 16), lambda i: (i, 0)),
            pl.BlockSpec((BN, 16), lambda i: (i, 0)),
        ],
        out_shape=[
            jax.ShapeDtypeStruct((H, N_pad, D), jnp.float32),
            jax.ShapeDtypeStruct((N_pad, 16), jnp.float32),
            jax.ShapeDtypeStruct((N_pad, 16), jnp.float32),
        ],
    )(U, esum, b, W, al, ar)


def _tc_final(U, esum, b, Wd1, bd1, Wd2, bd2):
    return pl.pallas_call(
        _tc_final_body,
        grid=(NBLK,),
        in_specs=[
            pl.BlockSpec((H, BN, D), lambda i: (0, i, 0)),
            pl.BlockSpec((2, BN, 16), lambda i: (0, i, 0)),
            pl.BlockSpec((H, D), lambda i: (0, 0)),
            pl.BlockSpec((HD, 512), lambda i: (0, 0)),
            pl.BlockSpec((1, 512), lambda i: (0, 0)),
            pl.BlockSpec((512, 1), lambda i: (0, 0)),
            pl.BlockSpec((1, 1), lambda i: (0, 0)),
        ],
        out_specs=[
            pl.BlockSpec((BN, 1), lambda i: (i, 0)),
            pl.BlockSpec((1, 1), lambda i: (0, 0)),
        ],
        out_shape=[
            jax.ShapeDtypeStruct((N_pad, 1), jnp.float32),
            jax.ShapeDtypeStruct((1, 1), jnp.float32),
        ],
    )(U, esum, b, Wd1, bd1, Wd2, bd2)


_MESH = plsc.VectorSubcoreMesh(core_axis_name="c", subcore_axis_name="s")
_SC_PARAMS = pltpu.CompilerParams(use_tc_tiling_on_sc=False)


@functools.partial(
    pl.kernel,
    mesh=_MESH,
    out_type=[
        jax.ShapeDtypeStruct((E_pad, 16), jnp.float32),
        jax.ShapeDtypeStruct((2 * N_pad, 16), jnp.float32),
    ],
    scratch_types=[
        pltpu.VMEM((4, 128), jnp.int32),
        pltpu.VMEM((4, 128), jnp.int32),
        pltpu.VMEM((128, 16), jnp.float32),
        pltpu.VMEM((128, 16), jnp.float32),
        pltpu.VMEM((128, 16), jnp.float32),
        pltpu.VMEM((ZROWS, 16), jnp.float32),
        pltpu.VMEM((32,), jnp.float32),
        pltpu.VMEM_SHARED((N_pad, 16), jnp.float32),
        pltpu.SemaphoreType.DMA,
        pltpu.SemaphoreType.DMA,
    ],
    compiler_params=_SC_PARAMS,
)
def _sc_edge_softmax(el_hbm, er_hbm, src_hbm, dst_hbm, tab_hbm, ee_out,
                     esum_out, sidx_v, didx_v, a_v, b_v, ee16_v, z_v, tab_v,
                     esum_sp, sem1, sem2):
    c = lax.axis_index("c")
    s = lax.axis_index("s")
    w = c * 16 + s
    z16 = jnp.zeros((16,), jnp.float32)

    pltpu.sync_copy(tab_hbm, tab_v)
    _fori(ZROWS, lambda r: z_v.__setitem__((r, slice(None)), z16))
    _fori(8, lambda p: pltpu.sync_copy(
        z_v, esum_sp.at[pl.ds(s * ROWS_T + p * ZROWS, ZROWS)]))
    plsc.subcore_barrier()

    def chunk(k):
        base = w * EPT + k * CH
        row0 = w * (EPT // 128) + k * 4
        pltpu.sync_copy(src_hbm.at[pl.ds(row0, 4)], sidx_v)
        pltpu.sync_copy(dst_hbm.at[pl.ds(row0, 4)], didx_v)

        def sub(j):
            cp1 = pltpu.async_copy(el_hbm.at[sidx_v.at[j]], a_v, sem1)
            cp2 = pltpu.async_copy(er_hbm.at[didx_v.at[j]], b_v, sem2)
            cp1.wait()
            cp2.wait()
            tab_a = tab_v[pl.ds(0, 16)]
            tab_b = tab_v[pl.ds(16, 16)]

            def grp(gg):
                for t in range(8):
                    g = gg * 8 + t
                    e = a_v[g, :] + b_v[g, :]
                    ee16_v[g, :] = _exp_precise(jnp.maximum(e, e * 0.2),
                                                tab_a, tab_b)
            _fori(16, grp)
            pltpu.sync_copy(ee16_v, esum_sp.at[didx_v.at[j]], add=True)
            pltpu.sync_copy(ee16_v, ee_out.at[pl.ds(base + j * 128, 128)])
        _fori(4, sub)
    _fori(NCHUNK, chunk)

    plsc.subcore_barrier()
    pltpu.sync_copy(esum_sp.at[pl.ds(s * ROWS_T, ROWS_T)],
                    esum_out.at[pl.ds(c * N_pad + s * ROWS_T, ROWS_T)])


@functools.partial(
    pl.kernel,
    mesh=_MESH,
    out_type=jax.ShapeDtypeStruct((H * N_pad, D), jnp.float32),
    scratch_types=[
        pltpu.VMEM((4, 128), jnp.int32),
        pltpu.VMEM((4, 128), jnp.int32),
        pltpu.VMEM((128, D), jnp.float32),
        pltpu.VMEM((128, D), jnp.float32),
        pltpu.VMEM((CH, 16), jnp.float32),
        pltpu.VMEM((ZROWS, D), jnp.float32),
        pltpu.VMEM_SHARED((N_pad, D), jnp.float32),
        pltpu.SemaphoreType.DMA,
    ],
    compiler_params=_SC_PARAMS,
)
def _sc_aggregate(feat_hbm, src_hbm, dst_hbm, ee_hbm, u_out,
                  sidx_v, didx_v, f_v, s_v, ee_v, z_v, acc_sp, sem1):
    c = lax.axis_index("c")
    s = lax.axis_index("s")
    ept = E_pad // 16
    nchunk = ept // CH
    z16 = jnp.zeros((16,), jnp.float32)

    def zrow(r):
        z_v[r, pl.ds(0, 16)] = z16
        z_v[r, pl.ds(16, 16)] = z16
    _fori(ZROWS, zrow)

    def head(hh):
        h = 2 * c + hh
        hidx = jnp.zeros((16,), jnp.int32) + h

        _fori(8, lambda p: pltpu.sync_copy(
            z_v, acc_sp.at[pl.ds(s * ROWS_T + p * ZROWS, ZROWS)]))
        plsc.subcore_barrier()

        def chunk(k):
            base = s * ept + k * CH
            row0 = s * (ept // 128) + k * 4
            pltpu.sync_copy(src_hbm.at[pl.ds(row0, 4)], sidx_v)
            pltpu.sync_copy(dst_hbm.at[pl.ds(row0, 4)], didx_v)
            off = h * N_pad

            def adj(t):
                jj = t // 8
                tt = t % 8
                sidx_v[jj, pl.ds(tt * 16, 16)] = (
                    sidx_v[jj, pl.ds(tt * 16, 16)] + off)
            _fori(32, adj)
            pltpu.sync_copy(ee_hbm.at[pl.ds(base, CH)], ee_v)

            def sub(j):
                pltpu.async_copy(feat_hbm.at[sidx_v.at[j]], f_v, sem1).wait()

                def grp(m):
                    for t in range(16):
                        e = m * 16 + t
                        eer = ee_v[j * 128 + e, :]
                        ab = eer.at[hidx].get(mode="promise_in_bounds")
                        s_v[e, pl.ds(0, 16)] = f_v[e, pl.ds(0, 16)] * ab
                        s_v[e, pl.ds(16, 16)] = f_v[e, pl.ds(16, 16)] * ab
                _fori(8, grp)
                pltpu.sync_copy(s_v, acc_sp.at[didx_v.at[j]], add=True)
            _fori(4, sub)
        _fori(nchunk, chunk)

        plsc.subcore_barrier()
        pltpu.sync_copy(acc_sp.at[pl.ds(s * ROWS_T, ROWS_T)],
                        u_out.at[pl.ds(h * N_pad + s * ROWS_T, ROWS_T)])
        plsc.subcore_barrier()
    _fori(2, head)


def kernel(seq, node_s, edge_index, W_s, W0, al0, ar0, b0, W1, al1, ar1, b1,
           W2, al2, ar2, b2, Wd1, bd1, Wd2, bd2):
    f32 = jnp.float32
    seq2 = jnp.concatenate(
        [seq.astype(jnp.int32), jnp.zeros((N_pad - N,), jnp.int32)]
    ).reshape(N_pad, 1)
    ns_p = jnp.concatenate(
        [node_s, jnp.zeros((N_pad - N, 6), f32)], axis=0)
    src = edge_index[0].astype(jnp.int32)
    dst = edge_index[1].astype(jnp.int32)
    pad = jnp.full((E_pad - E,), N, jnp.int32)
    srcR = jnp.concatenate([src, pad]).reshape(E_pad // 128, 128)
    dstR = jnp.concatenate([dst, pad]).reshape(E_pad // 128, 128)
    pow2_tabs = jnp.concatenate([
        jnp.exp2(8.0 * (jnp.arange(16, dtype=f32) - 8.0)),
        jnp.exp2(jnp.arange(16, dtype=f32)),
    ])

    feat, el, er = _tc_first(seq2, ns_p, W_s, W0,
                             al0.reshape(H, D), ar0.reshape(H, D))
    eet, esum = _sc_edge_softmax(el, er, srcR, dstR, pow2_tabs)
    U = _sc_aggregate(feat.reshape(H * N_pad, D), srcR, dstR, eet)

    for (W, al, ar, b_prev) in ((W1, al1, ar1, b0), (W2, al2, ar2, b1)):
        feat, el, er = _tc_mid(U.reshape(H, N_pad, D),
                               esum.reshape(2, N_pad, 16),
                               b_prev.reshape(H, D), W,
                               al.reshape(H, D), ar.reshape(H, D))
        eet, esum = _sc_edge_softmax(el, er, srcR, dstR, pow2_tabs)
        U = _sc_aggregate(feat.reshape(H * N_pad, D), srcR, dstR, eet)

    out_pad, graph = _tc_final(U.reshape(H, N_pad, D),
                               esum.reshape(2, N_pad, 16),
                               b2.reshape(H, D), Wd1, bd1.reshape(1, 512),
                               Wd2, bd2.reshape(1, 1))
    return out_pad[:N], graph


# speedup vs baseline: 34.3520x; 1.1919x over previous
import functools

import jax
import jax.numpy as jnp
from jax import lax
from jax.experimental import pallas as pl
from jax.experimental.pallas import tpu as pltpu
from jax.experimental.pallas import tpu_sc as plsc

N = 50000
E = 800000
H = 4
D = 32
HD = H * D

BN = 512
N_pad = 50176
E_pad = 819200
NBLK = N_pad // BN
NTILE = 32
EPT = E_pad // NTILE
CH = 512
NCHUNK = EPT // CH
ROWS_T = N_pad // 16
ZROWS = 392


def _fori(n, body):
    lax.fori_loop(0, n, lambda i, c: (body(i), 0)[1], 0)


_LOG2E = 1.4426950408889634
_LN2 = 0.6931471805599453
_RND = 12582912.0


def _exp_precise(x, tab_a, tab_b):
    x = jnp.minimum(jnp.maximum(x, -43.0), 43.0)
    t = x * _LOG2E
    nf = (t + _RND) - _RND
    z = (t - nf) * _LN2
    p = 1.0 / 5040.0
    for c in (1.0 / 720.0, 1.0 / 120.0, 1.0 / 24.0, 1.0 / 6.0, 0.5, 1.0, 1.0):
        p = p * z + c
    ni = nf.astype(jnp.int32)
    n1 = ni >> 3
    n2 = ni - (n1 << 3)
    ga = tab_a.at[n1 + 8].get(mode="promise_in_bounds")
    gb = tab_b.at[n2].get(mode="promise_in_bounds")
    return p * ga * gb


def _tc_first_body(seq_ref, ns_ref, ws_ref, w0_ref, al_ref, ar_ref,
                   feat_ref, el_ref, er_ref):
    t0 = jnp.dot(ws_ref[...], w0_ref[0:20, :], preferred_element_type=jnp.float32, precision=lax.Precision.HIGHEST)
    sv = seq_ref[...]
    iota = lax.broadcasted_iota(jnp.int32, (BN, 20), 1)
    oh = (sv == iota).astype(jnp.float32)
    feat = jnp.dot(oh, t0, preferred_element_type=jnp.float32, precision=lax.Precision.HIGHEST)
    feat = feat + jnp.dot(ns_ref[...], w0_ref[20:26, :],
                          preferred_element_type=jnp.float32, precision=lax.Precision.HIGHEST)
    el_cols = []
    er_cols = []
    for h in range(H):
        fh = feat[:, 32 * h:32 * h + 32]
        feat_ref[h] = fh
        el_cols.append(jnp.sum(fh * al_ref[h, :][None, :], axis=1, keepdims=True))
        er_cols.append(jnp.sum(fh * ar_ref[h, :][None, :], axis=1, keepdims=True))
    z12 = jnp.zeros((BN, 12), jnp.float32)
    el_ref[...] = jnp.concatenate(el_cols + [z12], axis=1)
    er_ref[...] = jnp.concatenate(er_cols + [z12], axis=1)


def _tc_mid_body(u_ref, es_ref, b_ref, w_ref, al_ref, ar_ref,
                 feat_ref, el_ref, er_ref):
    es = es_ref[0, :, 0:4] + es_ref[1, :, 0:4]
    esc = jnp.where(es == 0.0, 1.0, es)
    feat = jnp.zeros((BN, HD), jnp.float32)
    for h in range(H):
        xh = u_ref[h] / esc[:, h:h + 1] + b_ref[h, :][None, :]
        feat = feat + jnp.dot(xh, w_ref[32 * h:32 * h + 32, :],
                              preferred_element_type=jnp.float32, precision=lax.Precision.HIGHEST)
    el_cols = []
    er_cols = []
    for h in range(H):
        fh = feat[:, 32 * h:32 * h + 32]
        feat_ref[h] = fh
        el_cols.append(jnp.sum(fh * al_ref[h, :][None, :], axis=1, keepdims=True))
        er_cols.append(jnp.sum(fh * ar_ref[h, :][None, :], axis=1, keepdims=True))
    z12 = jnp.zeros((BN, 12), jnp.float32)
    el_ref[...] = jnp.concatenate(el_cols + [z12], axis=1)
    er_ref[...] = jnp.concatenate(er_cols + [z12], axis=1)


def _tc_final_body(u_ref, es_ref, b_ref, wd1_ref, bd1_ref, wd2_ref, bd2_ref,
                   out_ref, acc_ref):
    i = pl.program_id(0)
    es = es_ref[0, :, 0:4] + es_ref[1, :, 0:4]
    esc = jnp.where(es == 0.0, 1.0, es)
    y = jnp.zeros((BN, 512), jnp.float32)
    for h in range(H):
        xh = u_ref[h] / esc[:, h:h + 1] + b_ref[h, :][None, :]
        xh = jnp.maximum(xh, 0.0)
        y = y + jnp.dot(xh, wd1_ref[32 * h:32 * h + 32, :],
                        preferred_element_type=jnp.float32, precision=lax.Precision.HIGHEST)
    y = jnp.maximum(y + bd1_ref[...], 0.0)
    o = jnp.dot(y, wd2_ref[...], preferred_element_type=jnp.float32, precision=lax.Precision.HIGHEST)
    o = o + bd2_ref[...] + 0.5
    out_ref[...] = o

    @pl.when(i == 0)
    def _():
        acc_ref[...] = jnp.zeros_like(acc_ref)

    rows = lax.broadcasted_iota(jnp.int32, (BN, 1), 0) + i * BN
    om = jnp.where(rows < N, o, 0.0)
    acc_ref[...] += jnp.sum(om, axis=(0, 1), keepdims=False).reshape(1, 1) / N


def _tc_first(seq2, ns_p, W_s, W0, al, ar):
    return pl.pallas_call(
        _tc_first_body,
        grid=(NBLK,),
        in_specs=[
            pl.BlockSpec((BN, 1), lambda i: (i, 0)),
            pl.BlockSpec((BN, 6), lambda i: (i, 0)),
            pl.BlockSpec((20, 20), lambda i: (0, 0)),
            pl.BlockSpec((26, HD), lambda i: (0, 0)),
            pl.BlockSpec((H, D), lambda i: (0, 0)),
            pl.BlockSpec((H, D), lambda i: (0, 0)),
        ],
        out_specs=[
            pl.BlockSpec((H, BN, D), lambda i: (0, i, 0)),
            pl.BlockSpec((BN, 16), lambda i: (i, 0)),
            pl.BlockSpec((BN, 16), lambda i: (i, 0)),
        ],
        out_shape=[
            jax.ShapeDtypeStruct((H, N_pad, D), jnp.float32),
            jax.ShapeDtypeStruct((N_pad, 16), jnp.float32),
            jax.ShapeDtypeStruct((N_pad, 16), jnp.float32),
        ],
    )(seq2, ns_p, W_s, W0, al, ar)


def _tc_mid(U, esum, b, W, al, ar):
    return pl.pallas_call(
        _tc_mid_body,
        grid=(NBLK,),
        in_specs=[
            pl.BlockSpec((H, BN, D), lambda i: (0, i, 0)),
            pl.BlockSpec((2, BN, 16), lambda i: (0, i, 0)),
            pl.BlockSpec((H, D), lambda i: (0, 0)),
            pl.BlockSpec((HD, HD), lambda i: (0, 0)),
            pl.BlockSpec((H, D), lambda i: (0, 0)),
            pl.BlockSpec((H, D), lambda i: (0, 0)),
        ],
        out_specs=[
            pl.BlockSpec((H, BN, D), lambda i: (0, i, 0)),
            pl.BlockSpec((BN, 16), lambda i: (i, 0)),
            pl.BlockSpec((BN, 16), lambda i: (i, 0)),
        ],
        out_shape=[
            jax.ShapeDtypeStruct((H, N_pad, D), jnp.float32),
            jax.ShapeDtypeStruct((N_pad, 16), jnp.float32),
            jax.ShapeDtypeStruct((N_pad, 16), jnp.float32),
        ],
    )(U, esum, b, W, al, ar)


def _tc_final(U, esum, b, Wd1, bd1, Wd2, bd2):
    return pl.pallas_call(
        _tc_final_body,
        grid=(NBLK,),
        in_specs=[
            pl.BlockSpec((H, BN, D), lambda i: (0, i, 0)),
            pl.BlockSpec((2, BN, 16), lambda i: (0, i, 0)),
            pl.BlockSpec((H, D), lambda i: (0, 0)),
            pl.BlockSpec((HD, 512), lambda i: (0, 0)),
            pl.BlockSpec((1, 512), lambda i: (0, 0)),
            pl.BlockSpec((512, 1), lambda i: (0, 0)),
            pl.BlockSpec((1, 1), lambda i: (0, 0)),
        ],
        out_specs=[
            pl.BlockSpec((BN, 1), lambda i: (i, 0)),
            pl.BlockSpec((1, 1), lambda i: (0, 0)),
        ],
        out_shape=[
            jax.ShapeDtypeStruct((N_pad, 1), jnp.float32),
            jax.ShapeDtypeStruct((1, 1), jnp.float32),
        ],
    )(U, esum, b, Wd1, bd1, Wd2, bd2)


_MESH = plsc.VectorSubcoreMesh(core_axis_name="c", subcore_axis_name="s")
_SC_PARAMS = pltpu.CompilerParams(use_tc_tiling_on_sc=False)


@functools.partial(
    pl.kernel,
    mesh=_MESH,
    out_type=[
        jax.ShapeDtypeStruct((E_pad, 16), jnp.float32),
        jax.ShapeDtypeStruct((2 * N_pad, 16), jnp.float32),
    ],
    scratch_types=[
        pltpu.VMEM((4, 128), jnp.int32),
        pltpu.VMEM((4, 128), jnp.int32),
        pltpu.VMEM((CH, 16), jnp.float32),
        pltpu.VMEM((CH, 16), jnp.float32),
        pltpu.VMEM((CH, 16), jnp.float32),
        pltpu.VMEM((ZROWS, 16), jnp.float32),
        pltpu.VMEM((32,), jnp.float32),
        pltpu.VMEM_SHARED((N_pad, 16), jnp.float32),
        pltpu.SemaphoreType.DMA,
        pltpu.SemaphoreType.DMA,
        pltpu.SemaphoreType.DMA,
    ],
    compiler_params=_SC_PARAMS,
)
def _sc_edge_softmax(el_hbm, er_hbm, src_hbm, dst_hbm, tab_hbm, ee_out,
                     esum_out, sidx_v, didx_v, a_v, b_v, ee16_v, z_v, tab_v,
                     esum_sp, sem1, sem2, sem3):
    c = lax.axis_index("c")
    s = lax.axis_index("s")
    w = c * 16 + s
    z16 = jnp.zeros((16,), jnp.float32)

    pltpu.sync_copy(tab_hbm, tab_v)
    _fori(ZROWS, lambda r: z_v.__setitem__((r, slice(None)), z16))
    _fori(8, lambda p: pltpu.sync_copy(
        z_v, esum_sp.at[pl.ds(s * ROWS_T + p * ZROWS, ZROWS)]))
    plsc.subcore_barrier()

    def chunk(k):
        base = w * EPT + k * CH
        row0 = w * (EPT // 128) + k * 4
        ci1 = pltpu.async_copy(src_hbm.at[pl.ds(row0, 4)], sidx_v, sem2)
        ci2 = pltpu.async_copy(dst_hbm.at[pl.ds(row0, 4)], didx_v, sem2)
        ci1.wait()
        ci2.wait()
        cps = [pltpu.async_copy(el_hbm.at[sidx_v.at[j]],
                                a_v.at[pl.ds(j * 128, 128)], sem1)
               for j in range(4)]
        cps += [pltpu.async_copy(er_hbm.at[didx_v.at[j]],
                                 b_v.at[pl.ds(j * 128, 128)], sem1)
                for j in range(4)]
        for cp in cps:
            cp.wait()
        tab_a = tab_v[pl.ds(0, 16)]
        tab_b = tab_v[pl.ds(16, 16)]

        def grp(gg):
            for t in range(8):
                g = gg * 8 + t
                e = a_v[g, :] + b_v[g, :]
                ee16_v[g, :] = _exp_precise(jnp.maximum(e, e * 0.2),
                                            tab_a, tab_b)
        _fori(CH // 8, grp)
        cpe = pltpu.async_copy(ee16_v, ee_out.at[pl.ds(base, CH)], sem3)
        for j in range(4):
            pltpu.sync_copy(ee16_v.at[pl.ds(j * 128, 128)],
                            esum_sp.at[didx_v.at[j]], add=True)
        cpe.wait()
    _fori(NCHUNK, chunk)

    plsc.subcore_barrier()
    pltpu.sync_copy(esum_sp.at[pl.ds(s * ROWS_T, ROWS_T)],
                    esum_out.at[pl.ds(c * N_pad + s * ROWS_T, ROWS_T)])


@functools.partial(
    pl.kernel,
    mesh=_MESH,
    out_type=jax.ShapeDtypeStruct((H * N_pad, D), jnp.float32),
    scratch_types=[
        pltpu.VMEM((4, 128), jnp.int32),
        pltpu.VMEM((4, 128), jnp.int32),
        pltpu.VMEM((CH, D), jnp.float32),
        pltpu.VMEM((CH, 16), jnp.float32),
        pltpu.VMEM_SHARED((N_pad, D), jnp.float32),
        pltpu.SemaphoreType.DMA,
        pltpu.SemaphoreType.DMA,
        pltpu.SemaphoreType.DMA,
    ],
    compiler_params=_SC_PARAMS,
)
def _sc_aggregate(feat_hbm, src_hbm, dst_hbm, ee_hbm, u_out,
                  sidx_v, didx_v, f_v, ee_v, acc_sp,
                  sem1, sem2, sem3):
    c = lax.axis_index("c")
    s = lax.axis_index("s")
    ept = E_pad // 16
    nchunk = ept // CH
    zr = ROWS_T // 7
    z16 = jnp.zeros((16,), jnp.float32)

    def head(hh):
        h = 2 * c + hh
        hidx = jnp.zeros((16,), jnp.int32) + h

        def zrow(r):
            f_v[r, pl.ds(0, 16)] = z16
            f_v[r, pl.ds(16, 16)] = z16
        _fori(zr, zrow)
        _fori(7, lambda p: pltpu.sync_copy(
            f_v.at[pl.ds(0, zr)],
            acc_sp.at[pl.ds(s * ROWS_T + p * zr, zr)]))
        plsc.subcore_barrier()

        def chunk(k):
            base = s * ept + k * CH
            row0 = s * (ept // 128) + k * 4
            ci1 = pltpu.async_copy(src_hbm.at[pl.ds(row0, 4)], sidx_v, sem2)
            ci2 = pltpu.async_copy(dst_hbm.at[pl.ds(row0, 4)], didx_v, sem2)
            ci3 = pltpu.async_copy(ee_hbm.at[pl.ds(base, CH)], ee_v, sem2)
            ci1.wait()
            ci2.wait()
            ci3.wait()
            off = h * N_pad

            def adj(t):
                jj = t // 8
                tt = t % 8
                sidx_v[jj, pl.ds(tt * 16, 16)] = (
                    sidx_v[jj, pl.ds(tt * 16, 16)] + off)
            _fori(32, adj)
            cps = [pltpu.async_copy(feat_hbm.at[sidx_v.at[j]],
                                    f_v.at[pl.ds(j * 128, 128)], sem1)
                   for j in range(4)]
            for cp in cps:
                cp.wait()

            def grp(m):
                for t in range(16):
                    e = m * 16 + t
                    eer = ee_v[e, :]
                    ab = eer.at[hidx].get(mode="promise_in_bounds")
                    f_v[e, pl.ds(0, 16)] = f_v[e, pl.ds(0, 16)] * ab
                    f_v[e, pl.ds(16, 16)] = f_v[e, pl.ds(16, 16)] * ab
            _fori(CH // 16, grp)
            for j in range(4):
                pltpu.sync_copy(f_v.at[pl.ds(j * 128, 128)],
                                acc_sp.at[didx_v.at[j]], add=True)
        _fori(nchunk, chunk)

        plsc.subcore_barrier()
        pltpu.sync_copy(acc_sp.at[pl.ds(s * ROWS_T, ROWS_T)],
                        u_out.at[pl.ds(h * N_pad + s * ROWS_T, ROWS_T)])
        plsc.subcore_barrier()
    _fori(2, head)


def kernel(seq, node_s, edge_index, W_s, W0, al0, ar0, b0, W1, al1, ar1, b1,
           W2, al2, ar2, b2, Wd1, bd1, Wd2, bd2):
    f32 = jnp.float32
    seq2 = jnp.concatenate(
        [seq.astype(jnp.int32), jnp.zeros((N_pad - N,), jnp.int32)]
    ).reshape(N_pad, 1)
    ns_p = jnp.concatenate(
        [node_s, jnp.zeros((N_pad - N, 6), f32)], axis=0)
    src = edge_index[0].astype(jnp.int32)
    dst = edge_index[1].astype(jnp.int32)
    pad = jnp.full((E_pad - E,), N, jnp.int32)
    srcR = jnp.concatenate([src, pad]).reshape(E_pad // 128, 128)
    dstR = jnp.concatenate([dst, pad]).reshape(E_pad // 128, 128)
    pow2_tabs = jnp.concatenate([
        jnp.exp2(8.0 * (jnp.arange(16, dtype=f32) - 8.0)),
        jnp.exp2(jnp.arange(16, dtype=f32)),
    ])

    feat, el, er = _tc_first(seq2, ns_p, W_s, W0,
                             al0.reshape(H, D), ar0.reshape(H, D))
    eet, esum = _sc_edge_softmax(el, er, srcR, dstR, pow2_tabs)
    U = _sc_aggregate(feat.reshape(H * N_pad, D), srcR, dstR, eet)

    for (W, al, ar, b_prev) in ((W1, al1, ar1, b0), (W2, al2, ar2, b1)):
        feat, el, er = _tc_mid(U.reshape(H, N_pad, D),
                               esum.reshape(2, N_pad, 16),
                               b_prev.reshape(H, D), W,
                               al.reshape(H, D), ar.reshape(H, D))
        eet, esum = _sc_edge_softmax(el, er, srcR, dstR, pow2_tabs)
        U = _sc_aggregate(feat.reshape(H * N_pad, D), srcR, dstR, eet)

    out_pad, graph = _tc_final(U.reshape(H, N_pad, D),
                               esum.reshape(2, N_pad, 16),
                               b2.reshape(H, D), Wd1, bd1.reshape(1, 512),
                               Wd2, bd2.reshape(1, 1))
    return out_pad[:N], graph
